# baseline (speedup 1.0000x reference)
import functools

import jax
import jax.numpy as jnp
from jax import lax
from jax.experimental import pallas as pl
from jax.experimental.pallas import tpu as pltpu
from jax.experimental.pallas import tpu_sc as plsc

D_MODEL = 768
N_VOICES = 4
LANES = 16
CH = 64
NLC = D_MODEL // LANES


def _make_sc_kernel(n_rows, seq_len):
    info = plsc.get_sparse_core_info()
    nc, ns = info.num_cores, info.num_subcores
    nw = nc * ns
    rows_per_w = n_rows // nw

    mesh = plsc.VectorSubcoreMesh(core_axis_name="c", subcore_axis_name="s")

    @functools.partial(
        pl.kernel,
        mesh=mesh,
        out_type=jax.ShapeDtypeStruct((n_rows, D_MODEL), jnp.float32),
        scratch_types=[
            pltpu.VMEM((CH, D_MODEL), jnp.float32),
            pltpu.VMEM((CH // N_VOICES, D_MODEL), jnp.float32),
            pltpu.VMEM((N_VOICES, D_MODEL), jnp.float32),
        ],
    )
    def sc_kernel(x_hbm, ts_hbm, v_hbm, out_hbm, xb, tsb, vb):
        wid = lax.axis_index("s") * nc + lax.axis_index("c")
        base = wid * rows_per_w
        pltpu.sync_copy(v_hbm, vb)

        def chunk_body(it, carry):
            row0 = base + it * CH
            p0 = lax.rem(row0, seq_len)
            r_idx = pl.multiple_of(row0, CH)
            t_idx = pl.multiple_of(p0 // N_VOICES, CH // N_VOICES)
            pltpu.sync_copy(x_hbm.at[pl.ds(r_idx, CH)], xb)
            pltpu.sync_copy(ts_hbm.at[pl.ds(t_idx, CH // N_VOICES)], tsb)

            def t_body(t, c2):
                for l in range(NLC):
                    sl = pl.ds(l * LANES, LANES)
                    tv = tsb[t, sl]
                    for v in range(N_VOICES):
                        r = t * N_VOICES + v
                        xb[r, sl] = xb[r, sl] + (tv + vb[v, sl])
                return c2

            lax.fori_loop(0, CH // N_VOICES, t_body, 0)
            pltpu.sync_copy(xb, out_hbm.at[pl.ds(row0, CH)])
            return carry

        lax.fori_loop(0, rows_per_w // CH, chunk_body, 0)

    return sc_kernel


def kernel(x, timestep_emb, voice_emb):
    B, L, D = x.shape
    T = L // N_VOICES
    n_rows = B * L
    ts = timestep_emb[:T]
    xf = x.reshape(n_rows, D)
    out = _make_sc_kernel(n_rows, L)(xf, ts, voice_emb)
    return out.reshape(B, L, D)


# speedup vs baseline: 1.0289x; 1.0289x over previous
import functools

import jax
import jax.numpy as jnp
from jax import lax
from jax.experimental import pallas as pl
from jax.experimental.pallas import tpu as pltpu
from jax.experimental.pallas import tpu_sc as plsc

D_MODEL = 768
N_VOICES = 4
LANES = 16
CH = 128
NLC = D_MODEL // LANES


def _make_sc_kernel(n_rows, seq_len):
    info = plsc.get_sparse_core_info()
    nc, ns = info.num_cores, info.num_subcores
    nw = nc * ns
    rows_per_w = n_rows // nw

    mesh = plsc.VectorSubcoreMesh(core_axis_name="c", subcore_axis_name="s")

    @functools.partial(
        pl.kernel,
        mesh=mesh,
        out_type=jax.ShapeDtypeStruct((n_rows, D_MODEL), jnp.float32),
        scratch_types=[
            pltpu.VMEM((CH, D_MODEL), jnp.float32),
            pltpu.VMEM((CH // N_VOICES, D_MODEL), jnp.float32),
            pltpu.VMEM((N_VOICES, D_MODEL), jnp.float32),
        ],
    )
    def sc_kernel(x_hbm, ts_hbm, v_hbm, out_hbm, xb, tsb, vb):
        wid = lax.axis_index("s") * nc + lax.axis_index("c")
        base = wid * rows_per_w
        pltpu.sync_copy(v_hbm, vb)

        def chunk_body(it, carry):
            row0 = base + it * CH
            p0 = lax.rem(row0, seq_len)
            r_idx = pl.multiple_of(row0, CH)
            t_idx = pl.multiple_of(p0 // N_VOICES, CH // N_VOICES)
            pltpu.sync_copy(x_hbm.at[pl.ds(r_idx, CH)], xb)
            pltpu.sync_copy(ts_hbm.at[pl.ds(t_idx, CH // N_VOICES)], tsb)

            def t_body(t, c2):
                for l in range(NLC):
                    sl = pl.ds(l * LANES, LANES)
                    tv = tsb[t, sl]
                    for v in range(N_VOICES):
                        r = t * N_VOICES + v
                        xb[r, sl] = xb[r, sl] + (tv + vb[v, sl])
                return c2

            lax.fori_loop(0, CH // N_VOICES, t_body, 0)
            pltpu.sync_copy(xb, out_hbm.at[pl.ds(row0, CH)])
            return carry

        lax.fori_loop(0, rows_per_w // CH, chunk_body, 0)

    return sc_kernel


def kernel(x, timestep_emb, voice_emb):
    B, L, D = x.shape
    T = L // N_VOICES
    n_rows = B * L
    ts = timestep_emb[:T]
    xf = x.reshape(n_rows, D)
    out = _make_sc_kernel(n_rows, L)(xf, ts, voice_emb)
    return out.reshape(B, L, D)


# speedup vs baseline: 5.8378x; 5.6738x over previous
import jax
import jax.numpy as jnp
from jax.experimental import pallas as pl
from jax.experimental.pallas import tpu as pltpu

D_MODEL = 768
N_VOICES = 4


def _pe_add_kernel(ts_ref, v_ref, x_ref, o_ref, pe_ref):
    bt = ts_ref.shape[0]

    @pl.when(pl.program_id(1) == 0)
    def _build_pe():
        ts = ts_ref[...]
        t_pe = jnp.repeat(ts, N_VOICES, axis=0)
        v_pe = pltpu.repeat(v_ref[...], bt, axis=0)
        pe_ref[...] = t_pe + v_pe

    o_ref[...] = x_ref[...] + pe_ref[...][None]


def kernel(x, timestep_emb, voice_emb):
    B, L, D = x.shape
    T = L // N_VOICES
    ts = timestep_emb[:T]

    BT = 512
    BB = 2
    BL = BT * N_VOICES
    grid = (T // BT, B // BB)
    return pl.pallas_call(
        _pe_add_kernel,
        grid=grid,
        in_specs=[
            pl.BlockSpec((BT, D), lambda i, b: (i, 0)),
            pl.BlockSpec((N_VOICES, D), lambda i, b: (0, 0)),
            pl.BlockSpec((BB, BL, D), lambda i, b: (b, i, 0)),
        ],
        out_specs=pl.BlockSpec((BB, BL, D), lambda i, b: (b, i, 0)),
        out_shape=jax.ShapeDtypeStruct((B, L, D), x.dtype),
        scratch_shapes=[pltpu.VMEM((BL, D), jnp.float32)],
        compiler_params=pltpu.CompilerParams(
            vmem_limit_bytes=100 * 1024 * 1024,
        ),
    )(ts, voice_emb, x)
